# baseline (speedup 1.0000x reference)
import jax
import jax.numpy as jnp
import numpy as np
from functools import partial
from jax.experimental import pallas as pl

T = 3
N = 4096
E = 131072
X_DIM = 128
H_DIM = 128
Z_DIM = 64
ALPHA = 0.2

_ROWS = 256


def _nll_body(adj_ref, h1_ref, h2_ref, out_ref):
    t = pl.program_id(0)
    b = pl.program_id(1)

    @pl.when(jnp.logical_and(t == 0, b == 0))
    def _():
        out_ref[...] = jnp.zeros_like(out_ref)

    s = h1_ref[0] + h2_ref[0]
    p = jax.nn.sigmoid(s)
    p = jnp.clip(p, 1e-7, 1.0 - 1e-7)
    adj = adj_ref[0]
    term = adj * jnp.log(p) + (1.0 - adj) * jnp.log(1.0 - p)
    scale = 1.0 / (float(N) * float(N) * 8.0 * 128.0)
    out_ref[...] += jnp.sum(term) * scale


def _nll_all(adj, h1, h2):
    nb = N // _ROWS
    out = pl.pallas_call(
        _nll_body,
        grid=(T, nb),
        in_specs=[
            pl.BlockSpec((1, _ROWS, N), lambda t, b: (t, b, 0)),
            pl.BlockSpec((1, _ROWS, 1), lambda t, b: (t, b, 0)),
            pl.BlockSpec((1, 1, N), lambda t, b: (t, 0, 0)),
        ],
        out_specs=pl.BlockSpec((8, 128), lambda t, b: (0, 0)),
        out_shape=jax.ShapeDtypeStruct((8, 128), jnp.float32),
    )(adj, h1, h2)
    return -jnp.sum(out)


def _identity(v):
    return v


def _sp_gat(x, edge, W, a, act):
    n = x.shape[0]
    loops = jnp.arange(n, dtype=edge.dtype)
    e = jnp.concatenate([edge, jnp.stack([loops, loops])], axis=1)
    h = x @ W
    out_dim = W.shape[1]
    logits = h[e[0]] @ a[0, :out_dim] + h[e[1]] @ a[0, out_dim:]
    edge_e = jnp.exp(-jax.nn.leaky_relu(logits, negative_slope=ALPHA))
    rowsum = jax.ops.segment_sum(edge_e, e[0], num_segments=n)
    h_prime = jax.ops.segment_sum(edge_e[:, None] * h[e[1]], e[0], num_segments=n)
    return act(h_prime / rowsum[:, None])


def _kld_gauss(m1, s1, m2, s2):
    eps = 1e-8
    kld = 2.0 * jnp.log(s2 + eps) - 2.0 * jnp.log(s1 + eps) + (s1 ** 2 + (m1 - m2) ** 2) / ((s2 + eps) ** 2) - 1.0
    return (0.5 / m1.shape[0]) * jnp.sum(kld)


def kernel(x, edge_idx_list, adj_orig_dense_list, phi_x_W, phi_x_b, phi_z_W, phi_z_b, enc_W, enc_a, enc_mu_W, enc_mu_a, enc_lv_W, enc_lv_a, prior_W, prior_b, prior_mu_W, prior_mu_b, prior_lv_W, prior_lv_b, lstm_x_W, lstm_x_a, lstm_h_W, lstm_h_a, dec_a):
    h = jnp.zeros((N, H_DIM), dtype=jnp.float32)
    c = jnp.zeros((N, H_DIM), dtype=jnp.float32)
    kld = jnp.float32(0.0)
    h1s, h2s = [], []
    for t in range(T):
        edge = edge_idx_list[t]
        phi_x_t = jax.nn.relu(x[t] @ phi_x_W + phi_x_b)
        enc_in = jnp.concatenate([phi_x_t, h], axis=1)
        enc_t = _sp_gat(enc_in, edge, enc_W, enc_a, jax.nn.elu)
        enc_mu_t = _sp_gat(enc_t, edge, enc_mu_W, enc_mu_a, _identity)
        enc_std_t = _sp_gat(enc_t, edge, enc_lv_W, enc_lv_a, jax.nn.softplus)
        prior_t = jax.nn.elu(h @ prior_W + prior_b)
        prior_mu_t = prior_t @ prior_mu_W + prior_mu_b
        prior_std_t = jax.nn.softplus(prior_t @ prior_lv_W + prior_lv_b)
        eps = jax.random.normal(jax.random.fold_in(jax.random.key(7), t), enc_mu_t.shape, dtype=jnp.float32)
        z_t = enc_mu_t + eps * enc_std_t
        phi_z_t = jax.nn.relu(z_t @ phi_z_W + phi_z_b)
        x_lstm = jnp.concatenate([phi_x_t, phi_z_t], axis=1)
        ig = jax.nn.sigmoid(_sp_gat(x_lstm, edge, lstm_x_W[0], lstm_x_a[0], _identity) + _sp_gat(h, edge, lstm_h_W[0], lstm_h_a[0], _identity))
        fg = jax.nn.sigmoid(_sp_gat(x_lstm, edge, lstm_x_W[1], lstm_x_a[1], _identity) + _sp_gat(h, edge, lstm_h_W[1], lstm_h_a[1], _identity))
        og = jax.nn.sigmoid(_sp_gat(x_lstm, edge, lstm_x_W[2], lstm_x_a[2], _identity) + _sp_gat(h, edge, lstm_h_W[2], lstm_h_a[2], _identity))
        ct = jnp.tanh(_sp_gat(x_lstm, edge, lstm_x_W[3], lstm_x_a[3], _identity) + _sp_gat(h, edge, lstm_h_W[3], lstm_h_a[3], _identity))
        c = fg * c + ig * ct
        h = og * jnp.tanh(c)
        kld = kld + _kld_gauss(enc_mu_t, enc_std_t, prior_mu_t, prior_std_t)
        h1s.append(z_t @ dec_a[:Z_DIM, :])
        h2s.append((z_t @ dec_a[Z_DIM:, :]).T)
    h1 = jnp.stack(h1s)
    h2 = jnp.stack(h2s)
    nll = _nll_all(adj_orig_dense_list, h1, h2)
    return jnp.stack([kld, nll])


# speedup vs baseline: 15.2541x; 15.2541x over previous
import jax
import jax.numpy as jnp
import numpy as np
from functools import partial
from jax.experimental import pallas as pl
from jax.experimental.pallas import tpu as pltpu

T = 3
N = 4096
E = 131072
X_DIM = 128
H_DIM = 128
Z_DIM = 64
ALPHA = 0.2

_R = 512
_C = 1024
_ROWS = 256


def _gat_body(acts, m_ref, ht_ref, si_ref, sj_ref, out_ref, rs_ref):
    cb = pl.program_id(1)
    ncb = pl.num_programs(1)
    G = ht_ref.shape[0]

    @pl.when(cb == 0)
    def _():
        out_ref[...] = jnp.zeros_like(out_ref)
        rs_ref[...] = jnp.zeros_like(rs_ref)

    m = m_ref[...]
    for k in range(G):
        s = si_ref[k] + sj_ref[k]
        w = jnp.exp(jnp.where(s > 0, -s, -ALPHA * s))
        p = m * w
        out_ref[k] += jnp.dot(p, ht_ref[k], preferred_element_type=jnp.float32)
        rs_ref[k] += jnp.sum(p, axis=1, keepdims=True)

    @pl.when(cb == ncb - 1)
    def _():
        for k in range(G):
            out_ref[k] = acts[k](out_ref[k] / rs_ref[k])


def _gat_pass(M, HT, SI, SJ, acts):
    G, n, D = HT.shape
    grid = (N // _R, N // _C)
    return pl.pallas_call(
        partial(_gat_body, acts),
        grid=grid,
        in_specs=[
            pl.BlockSpec((_R, _C), lambda rb, cb: (rb, cb)),
            pl.BlockSpec((G, _C, D), lambda rb, cb: (0, cb, 0)),
            pl.BlockSpec((G, _R, 1), lambda rb, cb: (0, rb, 0)),
            pl.BlockSpec((G, 1, _C), lambda rb, cb: (0, 0, cb)),
        ],
        out_specs=pl.BlockSpec((G, _R, D), lambda rb, cb: (0, rb, 0)),
        out_shape=jax.ShapeDtypeStruct((G, N, D), jnp.float32),
        scratch_shapes=[pltpu.VMEM((G, _R, 1), jnp.float32)],
    )(M, HT, SI, SJ)


def _scores(HT, A):
    G, n, D = HT.shape
    si = jnp.einsum('gnd,gd->gn', HT, A[:, :D])
    sj = jnp.einsum('gnd,gd->gn', HT, A[:, D:])
    return si[:, :, None], sj[:, None, :]


def _nll_body(adj_ref, h1_ref, h2_ref, out_ref):
    t = pl.program_id(0)
    b = pl.program_id(1)

    @pl.when(jnp.logical_and(t == 0, b == 0))
    def _():
        out_ref[...] = jnp.zeros_like(out_ref)

    s = h1_ref[0] + h2_ref[0]
    p = jax.nn.sigmoid(s)
    p = jnp.clip(p, 1e-7, 1.0 - 1e-7)
    adj = adj_ref[0]
    term = adj * jnp.log(p) + (1.0 - adj) * jnp.log(1.0 - p)
    scale = 1.0 / (float(N) * float(N) * 8.0 * 128.0)
    out_ref[...] += jnp.sum(term) * scale


def _nll_all(adj, h1, h2):
    nb = N // _ROWS
    out = pl.pallas_call(
        _nll_body,
        grid=(T, nb),
        in_specs=[
            pl.BlockSpec((1, _ROWS, N), lambda t, b: (t, b, 0)),
            pl.BlockSpec((1, _ROWS, 1), lambda t, b: (t, b, 0)),
            pl.BlockSpec((1, 1, N), lambda t, b: (t, 0, 0)),
        ],
        out_specs=pl.BlockSpec((8, 128), lambda t, b: (0, 0)),
        out_shape=jax.ShapeDtypeStruct((8, 128), jnp.float32),
    )(adj, h1, h2)
    return -jnp.sum(out)


def _identity(v):
    return v


def _elu(v):
    return jnp.where(v > 0, v, jnp.exp(jnp.minimum(v, 0.0)) - 1.0)


def _softplus(v):
    return jnp.maximum(v, 0.0) + jnp.log(1.0 + jnp.exp(-jnp.abs(v)))


def _kld_gauss(m1, s1, m2, s2):
    eps = 1e-8
    kld = 2.0 * jnp.log(s2 + eps) - 2.0 * jnp.log(s1 + eps) + (s1 ** 2 + (m1 - m2) ** 2) / ((s2 + eps) ** 2) - 1.0
    return (0.5 / m1.shape[0]) * jnp.sum(kld)


def _build_m(edge):
    m = jnp.zeros((N, N), dtype=jnp.float32)
    m = m.at[edge[0], edge[1]].add(1.0)
    ar = jnp.arange(N)
    return m.at[ar, ar].add(1.0)


def kernel(x, edge_idx_list, adj_orig_dense_list, phi_x_W, phi_x_b, phi_z_W, phi_z_b, enc_W, enc_a, enc_mu_W, enc_mu_a, enc_lv_W, enc_lv_a, prior_W, prior_b, prior_mu_W, prior_mu_b, prior_lv_W, prior_lv_b, lstm_x_W, lstm_x_a, lstm_h_W, lstm_h_a, dec_a):
    h = jnp.zeros((N, H_DIM), dtype=jnp.float32)
    c = jnp.zeros((N, H_DIM), dtype=jnp.float32)
    kld = jnp.float32(0.0)
    h1s, h2s = [], []
    for t in range(T):
        M = _build_m(edge_idx_list[t])
        phi_x_t = jax.nn.relu(x[t] @ phi_x_W + phi_x_b)

        h_enc = jnp.concatenate([phi_x_t, h], axis=1) @ enc_W
        HT1 = h_enc[None]
        SI1, SJ1 = _scores(HT1, enc_a)
        enc_t = _gat_pass(M, HT1, SI1, SJ1, [_elu])[0]

        HT2 = jnp.stack([enc_t @ enc_mu_W, enc_t @ enc_lv_W])
        A2 = jnp.stack([enc_mu_a[0], enc_lv_a[0]])
        SI2, SJ2 = _scores(HT2, A2)
        o2 = _gat_pass(M, HT2, SI2, SJ2, [_identity, _softplus])
        enc_mu_t, enc_std_t = o2[0], o2[1]

        prior_t = jax.nn.elu(h @ prior_W + prior_b)
        prior_mu_t = prior_t @ prior_mu_W + prior_mu_b
        prior_std_t = jax.nn.softplus(prior_t @ prior_lv_W + prior_lv_b)
        eps = jax.random.normal(jax.random.fold_in(jax.random.key(7), t), enc_mu_t.shape, dtype=jnp.float32)
        z_t = enc_mu_t + eps * enc_std_t
        phi_z_t = jax.nn.relu(z_t @ phi_z_W + phi_z_b)
        x_lstm = jnp.concatenate([phi_x_t, phi_z_t], axis=1)

        HT3 = jnp.concatenate([
            jnp.einsum('nk,gkd->gnd', x_lstm, lstm_x_W),
            jnp.einsum('nk,gkd->gnd', h, lstm_h_W),
        ])
        A3 = jnp.concatenate([lstm_x_a[:, 0, :], lstm_h_a[:, 0, :]])
        SI3, SJ3 = _scores(HT3, A3)
        o3 = _gat_pass(M, HT3, SI3, SJ3, [_identity] * 8)

        ig = jax.nn.sigmoid(o3[0] + o3[4])
        fg = jax.nn.sigmoid(o3[1] + o3[5])
        og = jax.nn.sigmoid(o3[2] + o3[6])
        ct = jnp.tanh(o3[3] + o3[7])
        c = fg * c + ig * ct
        h = og * jnp.tanh(c)
        kld = kld + _kld_gauss(enc_mu_t, enc_std_t, prior_mu_t, prior_std_t)
        h1s.append(z_t @ dec_a[:Z_DIM, :])
        h2s.append((z_t @ dec_a[Z_DIM:, :]).T)

    h1 = jnp.stack(h1s)
    h2 = jnp.stack(h2s)
    nll = _nll_all(adj_orig_dense_list, h1, h2)
    return jnp.stack([kld, nll])
